# baseline (speedup 1.0000x reference)
import functools

import jax
import jax.numpy as jnp
import numpy as np
from jax import lax
from jax.experimental import pallas as pl
from jax.experimental.pallas import tpu as pltpu
from jax.experimental.pallas import tpu_sc as plsc

H = 64
W = 64
N = H * W
R = 3.0
EPS = 0.01
PPW = 256
NCH = 16


def _clipx(v):
    return jnp.minimum(jnp.maximum(v, 0.0), float(W - 1))


def _clipy(v):
    return jnp.minimum(jnp.maximum(v, 0.0), float(H - 1))


def _bilinear_prep(xv, yv):
    x = _clipx(xv)
    y = _clipy(yv)
    x0 = x.astype(jnp.int32)
    y0 = y.astype(jnp.int32)
    wx = x - x0.astype(jnp.float32)
    wy = y - y0.astype(jnp.float32)
    x1 = jnp.minimum(x0 + 1, W - 1)
    y1 = jnp.minimum(y0 + 1, H - 1)
    return x0, y0, x1, y1, wx, wy


def _bilinear_mix(v00, v01, v10, v11, wx, wy):
    return (v00 * (1 - wx) * (1 - wy) + v01 * wx * (1 - wy)
            + v10 * (1 - wx) * wy + v11 * wx * wy)


def _propagate_body(m_hbm, corr_hbm, noise_hbm, res_hbm,
                    xbuf, ybuf, sbuf, cxb, cyb, wxb, wyb,
                    stx, sty, nzv, idxb, taps, xs, ys, sem):
    c = lax.axis_index("c")
    s = lax.axis_index("s")
    base = s * PPW
    fwd = c == 0

    pltpu.sync_copy(m_hbm.at[c, 0, pl.ds(base, PPW)], xbuf)
    pltpu.sync_copy(m_hbm.at[c, 1, pl.ds(base, PPW)], ybuf)
    for t in range(6):
        pltpu.sync_copy(noise_hbm.at[c, t, pl.ds(base, PPW)], nzv.at[t])

    fsel = fwd.astype(jnp.int32)

    def eval_cand(init):
        def build_chunk(k):
            sl = pl.ds(k * 16, 16)
            xv = cxb[sl]
            yv = cyb[sl]
            x0, y0, x1, y1, wx, wy = _bilinear_prep(xv, yv)
            wxb[sl] = wx
            wyb[sl] = wy
            pv = base + k * 16 + lax.iota(jnp.int32, 16)
            half = k >> 3
            colo = (k & 7) * 16
            for tap, (yy, xx) in enumerate(((y0, x0), (y0, x1),
                                            (y1, x0), (y1, x1))):
                b = yy * W + xx
                addr = (pv * N + b) * fsel + (b * N + pv) * (1 - fsel)
                idxb[2 * tap + half, pl.ds(colo, 16)] = addr

        copies = []
        for half in range(2):
            def build2(i, _):
                build_chunk(half * 8 + i * 2)
                build_chunk(half * 8 + i * 2 + 1)
                return 0

            lax.fori_loop(0, 4, build2, 0)
            copies += [
                pltpu.async_copy(corr_hbm.at[idxb.at[2 * tap + half]],
                                 taps.at[2 * tap + half], sem)
                for tap in range(4)
            ]
        for cp in copies:
            cp.wait()

        def combine_chunk(k):
            sl = pl.ds(k * 16, 16)
            half = k >> 3
            colo = pl.ds((k & 7) * 16, 16)
            v00 = taps[0 + half, colo]
            v01 = taps[2 + half, colo]
            v10 = taps[4 + half, colo]
            v11 = taps[6 + half, colo]
            val = _bilinear_mix(v00, v01, v10, v11, wxb[sl], wyb[sl])
            if init:
                sbuf[sl] = val
            else:
                cur = sbuf[sl]
                better = val > cur
                xbuf[sl] = jnp.where(better, cxb[sl], xbuf[sl])
                ybuf[sl] = jnp.where(better, cyb[sl], ybuf[sl])
                sbuf[sl] = jnp.where(better, val, cur)

        def combine2(i, _):
            combine_chunk(i * 2)
            combine_chunk(i * 2 + 1)
            return 0

        lax.fori_loop(0, NCH // 2, combine2, 0)

    def seed_cand(i, _):
        for j in range(2):
            sl = pl.ds((i * 2 + j) * 16, 16)
            cxb[sl] = xbuf[sl]
            cyb[sl] = ybuf[sl]
        return 0

    lax.fori_loop(0, NCH // 2, seed_cand, 0)
    eval_cand(init=True)

    def round_body(r, _):
        neg_d0 = jnp.logical_or(r == 1, r == 2)
        d0 = jnp.where(neg_d0, -1, 1)
        d1 = jnp.where(r & 1 == 1, -1, 1)
        d0f = d0.astype(jnp.float32)
        d1f = d1.astype(jnp.float32)

        plsc.subcore_barrier()
        pltpu.sync_copy(xbuf, xs.at[pl.ds(64 + base, PPW)])
        pltpu.sync_copy(ybuf, ys.at[pl.ds(64 + base, PPW)])

        @pl.when(s == 0)
        def _():
            pltpu.sync_copy(xbuf.at[pl.ds(0, 64)], xs.at[pl.ds(65 * 64, 64)])
            pltpu.sync_copy(ybuf.at[pl.ds(0, 64)], ys.at[pl.ds(65 * 64, 64)])

        @pl.when(s == 15)
        def _():
            pltpu.sync_copy(xbuf.at[pl.ds(192, 64)], xs.at[pl.ds(0, 64)])
            pltpu.sync_copy(ybuf.at[pl.ds(192, 64)], ys.at[pl.ds(0, 64)])

        plsc.subcore_barrier()

        start = (4 * s - d0 + 1) * 64
        pltpu.sync_copy(xs.at[pl.ds(start, PPW)], stx)
        pltpu.sync_copy(ys.at[pl.ds(start, PPW)], sty)

        def row_cand(k, _):
            sl = pl.ds(k * 16, 16)
            cxb[sl] = _clipx(stx[sl])
            cyb[sl] = _clipy(sty[sl] + d0f)
            return 0

        lax.fori_loop(0, NCH, row_cand, 0)
        eval_cand(init=False)

        def col_cand(k, _):
            sl = pl.ds(k * 16, 16)
            rb = (k >> 2) * 64
            kc = k & 3
            prev_sl = pl.ds(rb + ((kc - 1) & 3) * 16, 16)
            next_sl = pl.ds(rb + ((kc + 1) & 3) * 16, 16)
            d1p = (d1 > 0).astype(jnp.float32)
            lv = lax.iota(jnp.int32, 16)
            perm = (lv - d1) & 15

            dnums = lax.GatherDimensionNumbers(
                offset_dims=(), collapsed_slice_dims=(0,),
                start_index_map=(0,))

            def vperm(vec):
                return lax.gather(
                    vec, perm[:, None], dnums, (1,),
                    mode=lax.GatherScatterMode.PROMISE_IN_BOUNDS)

            def shifted(buf):
                rc = vperm(buf[sl])
                rp = vperm(buf[prev_sl])
                rn = vperm(buf[next_sl])
                right = jnp.where(lv == 0, rp, rc)
                left = jnp.where(lv == 15, rn, rc)
                return right * d1p + left * (1.0 - d1p)

            cxb[sl] = _clipx(shifted(xbuf) + d1f)
            cyb[sl] = _clipy(shifted(ybuf))
            return 0

        lax.fori_loop(0, NCH, col_cand, 0)
        eval_cand(init=False)

        @pl.when(r < 3)
        def _():
            def rs_cand(k, _):
                sl = pl.ds(k * 16, 16)
                cxb[sl] = _clipx(xbuf[sl] + nzv[2 * r, sl])
                cyb[sl] = _clipy(ybuf[sl] + nzv[2 * r + 1, sl])
                return 0

            lax.fori_loop(0, NCH, rs_cand, 0)
            eval_cand(init=False)

        return 0

    lax.fori_loop(0, 4, round_body, 0)

    pltpu.sync_copy(xbuf, res_hbm.at[c, 0, pl.ds(base, PPW)])
    pltpu.sync_copy(ybuf, res_hbm.at[c, 1, pl.ds(base, PPW)])


def _fusion_body(res_flat_hbm, mf_hbm, out_hbm,
                 rfx, rfy, mfx, mfy, ox, oy, wxb, wyb, idxb, taps, sem):
    c = lax.axis_index("c")
    s = lax.axis_index("s")
    wid = s * 2 + c
    base = wid * 128

    pltpu.sync_copy(res_flat_hbm.at[pl.ds(base, 128)], rfx)
    pltpu.sync_copy(res_flat_hbm.at[pl.ds(N + base, 128)], rfy)
    pltpu.sync_copy(mf_hbm.at[0, pl.ds(base, 128)], mfx)
    pltpu.sync_copy(mf_hbm.at[1, pl.ds(base, 128)], mfy)

    def build(k, _):
        sl = pl.ds(k * 16, 16)
        x0, y0, x1, y1, wx, wy = _bilinear_prep(rfx[sl], rfy[sl])
        wxb[sl] = wx
        wyb[sl] = wy
        colo = pl.ds(k * 16, 16)
        for tap, (yy, xx) in enumerate(((y0, x0), (y0, x1),
                                        (y1, x0), (y1, x1))):
            i = yy * W + xx
            idxb[tap, colo] = 2 * N + i
            idxb[4 + tap, colo] = 3 * N + i
        return 0

    lax.fori_loop(0, 8, build, 0)

    copies = [pltpu.async_copy(res_flat_hbm.at[idxb.at[r]], taps.at[r], sem)
              for r in range(8)]
    for cp in copies:
        cp.wait()

    def chunk(k, _):
        sl = pl.ds(k * 16, 16)
        fx = rfx[sl]
        fy = rfy[sl]
        wx = wxb[sl]
        wy = wyb[sl]
        cx = _bilinear_mix(taps[0, sl], taps[1, sl], taps[2, sl],
                           taps[3, sl], wx, wy)
        cy = _bilinear_mix(taps[4, sl], taps[5, sl], taps[6, sl],
                           taps[7, sl], wx, wy)
        diff = jnp.maximum(jnp.abs(fx - cx), jnp.abs(fy - cy))
        invalid = diff > EPS
        ox[sl] = jnp.where(invalid, mfx[sl], fx)
        oy[sl] = jnp.where(invalid, mfy[sl], fy)
        return 0

    lax.fori_loop(0, 8, chunk, 0)

    pltpu.sync_copy(ox, out_hbm.at[0, pl.ds(base, 128)])
    pltpu.sync_copy(oy, out_hbm.at[1, pl.ds(base, 128)])


def _make_noise():
    key = jax.random.key(42)
    planes = []
    for h in range(2):
        kh = jax.random.fold_in(key, h)
        for t in range(3):
            n = R * jax.random.normal(jax.random.fold_in(kh, t),
                                      (1, H, W, 2), dtype=jnp.float32)
            n2 = n.reshape(N, 2).T
            planes.append(n2)
    return jnp.concatenate(planes).reshape(2, 6, N)


_NOISE_CACHE = []


def _noise_const():
    if not _NOISE_CACHE:
        try:
            _NOISE_CACHE.append(np.asarray(_make_noise()))
        except Exception:
            return _make_noise()
    return jnp.asarray(_NOISE_CACHE[0])


def kernel(matching_f, matching_b, corr_map):
    mesh = plsc.VectorSubcoreMesh(core_axis_name="c", subcore_axis_name="s")

    propagate = functools.partial(
        pl.kernel,
        mesh=mesh,
        out_type=jax.ShapeDtypeStruct((2, 2, N), jnp.float32),
        scratch_types=[
            pltpu.VMEM((PPW,), jnp.float32),
            pltpu.VMEM((PPW,), jnp.float32),
            pltpu.VMEM((PPW,), jnp.float32),
            pltpu.VMEM((PPW,), jnp.float32),
            pltpu.VMEM((PPW,), jnp.float32),
            pltpu.VMEM((PPW,), jnp.float32),
            pltpu.VMEM((PPW,), jnp.float32),
            pltpu.VMEM((PPW,), jnp.float32),
            pltpu.VMEM((PPW,), jnp.float32),
            pltpu.VMEM((6, PPW), jnp.float32),
            pltpu.VMEM((8, 128), jnp.int32),
            pltpu.VMEM((8, 128), jnp.float32),
            pltpu.VMEM_SHARED((66 * 64,), jnp.float32),
            pltpu.VMEM_SHARED((66 * 64,), jnp.float32),
            pltpu.SemaphoreType.DMA,
        ],
    )(_propagate_body)

    fusion = functools.partial(
        pl.kernel,
        mesh=mesh,
        out_type=jax.ShapeDtypeStruct((2, N), jnp.float32),
        scratch_types=[
            pltpu.VMEM((128,), jnp.float32),
            pltpu.VMEM((128,), jnp.float32),
            pltpu.VMEM((128,), jnp.float32),
            pltpu.VMEM((128,), jnp.float32),
            pltpu.VMEM((128,), jnp.float32),
            pltpu.VMEM((128,), jnp.float32),
            pltpu.VMEM((128,), jnp.float32),
            pltpu.VMEM((128,), jnp.float32),
            pltpu.VMEM((8, 128), jnp.int32),
            pltpu.VMEM((8, 128), jnp.float32),
            pltpu.SemaphoreType.DMA,
        ],
    )(_fusion_body)

    m_init = jnp.stack([matching_f.reshape(2, N), matching_b.reshape(2, N)])
    corr_flat = corr_map.reshape(N * N)
    noise = _noise_const()

    res = propagate(m_init, corr_flat, noise)
    out = fusion(res.reshape(4 * N), matching_f.reshape(2, N))
    return out.reshape(1, 2, H, W)


# speedup vs baseline: 1.5566x; 1.5566x over previous
import functools

import jax
import jax.numpy as jnp
import numpy as np
from jax import lax
from jax.experimental import pallas as pl
from jax.experimental.pallas import tpu as pltpu
from jax.experimental.pallas import tpu_sc as plsc

H = 64
W = 64
N = H * W
R = 3.0
EPS = 0.01
PPW = 256
NCH = 16


def _clipx(v):
    return jnp.minimum(jnp.maximum(v, 0.0), float(W - 1))


def _clipy(v):
    return jnp.minimum(jnp.maximum(v, 0.0), float(H - 1))


def _bilinear_prep(xv, yv):
    x = _clipx(xv)
    y = _clipy(yv)
    x0 = x.astype(jnp.int32)
    y0 = y.astype(jnp.int32)
    wx = x - x0.astype(jnp.float32)
    wy = y - y0.astype(jnp.float32)
    x1 = jnp.minimum(x0 + 1, W - 1)
    y1 = jnp.minimum(y0 + 1, H - 1)
    return x0, y0, x1, y1, wx, wy


def _bilinear_mix(v00, v01, v10, v11, wx, wy):
    return (v00 * (1 - wx) * (1 - wy) + v01 * wx * (1 - wy)
            + v10 * (1 - wx) * wy + v11 * wx * wy)


def _propagate_body(m_hbm, corr_hbm, noise_hbm, res_hbm,
                    xbuf, ybuf, sbuf, cxb, cyb, wxb, wyb,
                    stx, sty, nzv, idxb, taps, xs, ys, sem):
    c = lax.axis_index("c")
    s = lax.axis_index("s")
    base = s * PPW
    fwd = c == 0

    init_copies = [
        pltpu.async_copy(m_hbm.at[c, 0, pl.ds(base, PPW)], xbuf, sem),
        pltpu.async_copy(m_hbm.at[c, 1, pl.ds(base, PPW)], ybuf, sem),
    ] + [
        pltpu.async_copy(noise_hbm.at[c, t, pl.ds(base, PPW)], nzv.at[t], sem)
        for t in range(6)
    ]
    for cp in init_copies:
        cp.wait()

    fsel = fwd.astype(jnp.int32)

    def eval_cand(init):
        def build_chunk(k):
            sl = pl.ds(k * 16, 16)
            xv = cxb[sl]
            yv = cyb[sl]
            x0, y0, x1, y1, wx, wy = _bilinear_prep(xv, yv)
            wxb[sl] = wx
            wyb[sl] = wy
            pv = base + k * 16 + lax.iota(jnp.int32, 16)
            half = k >> 3
            colo = (k & 7) * 16
            iv = pv >> 6
            jv = pv & (W - 1)
            pperm = ((iv & 31) << 7) + ((iv >> 5) << 6) + jv
            a0 = ((y0 & 31) << 7) + ((y0 >> 5) << 6)
            a1 = ((y1 & 31) << 7) + ((y1 >> 5) << 6)
            for tap, (ay, yy, xx) in enumerate(((a0, y0, x0), (a0, y0, x1),
                                                (a1, y1, x0), (a1, y1, x1))):
                t_perm = ay + xx
                t_plain = yy * W + xx
                addr = ((pv * N + t_perm) * fsel
                        + (t_plain * N + pperm) * (1 - fsel))
                idxb[2 * tap + half, pl.ds(colo, 16)] = addr

        copies = []
        for half in range(2):
            def build2(i, _):
                build_chunk(half * 8 + i * 2)
                build_chunk(half * 8 + i * 2 + 1)
                return 0

            lax.fori_loop(0, 4, build2, 0)
            copies += [
                pltpu.async_copy(corr_hbm.at[idxb.at[2 * tap + half]],
                                 taps.at[2 * tap + half], sem)
                for tap in range(4)
            ]
        for cp in copies:
            cp.wait()

        def combine_chunk(k):
            sl = pl.ds(k * 16, 16)
            half = k >> 3
            colo = pl.ds((k & 7) * 16, 16)
            v00 = taps[0 + half, colo]
            v01 = taps[2 + half, colo]
            v10 = taps[4 + half, colo]
            v11 = taps[6 + half, colo]
            val = _bilinear_mix(v00, v01, v10, v11, wxb[sl], wyb[sl])
            if init:
                sbuf[sl] = val
            else:
                cur = sbuf[sl]
                better = val > cur
                xbuf[sl] = jnp.where(better, cxb[sl], xbuf[sl])
                ybuf[sl] = jnp.where(better, cyb[sl], ybuf[sl])
                sbuf[sl] = jnp.where(better, val, cur)

        def combine2(i, _):
            combine_chunk(i * 2)
            combine_chunk(i * 2 + 1)
            return 0

        lax.fori_loop(0, NCH // 2, combine2, 0)

    def seed_cand(i, _):
        for j in range(2):
            sl = pl.ds((i * 2 + j) * 16, 16)
            cxb[sl] = xbuf[sl]
            cyb[sl] = ybuf[sl]
        return 0

    lax.fori_loop(0, NCH // 2, seed_cand, 0)
    eval_cand(init=True)

    def round_body(r, _):
        neg_d0 = jnp.logical_or(r == 1, r == 2)
        d0 = jnp.where(neg_d0, -1, 1)
        d1 = jnp.where(r & 1 == 1, -1, 1)
        d0f = d0.astype(jnp.float32)
        d1f = d1.astype(jnp.float32)

        plsc.subcore_barrier()
        pubs = [pltpu.async_copy(xbuf, xs.at[pl.ds(64 + base, PPW)], sem),
                pltpu.async_copy(ybuf, ys.at[pl.ds(64 + base, PPW)], sem)]

        @pl.when(s == 0)
        def _():
            halo = [pltpu.async_copy(xbuf.at[pl.ds(0, 64)],
                                     xs.at[pl.ds(65 * 64, 64)], sem),
                    pltpu.async_copy(ybuf.at[pl.ds(0, 64)],
                                     ys.at[pl.ds(65 * 64, 64)], sem)]
            for cp in halo:
                cp.wait()

        @pl.when(s == 15)
        def _():
            halo = [pltpu.async_copy(xbuf.at[pl.ds(192, 64)],
                                     xs.at[pl.ds(0, 64)], sem),
                    pltpu.async_copy(ybuf.at[pl.ds(192, 64)],
                                     ys.at[pl.ds(0, 64)], sem)]
            for cp in halo:
                cp.wait()

        for cp in pubs:
            cp.wait()
        plsc.subcore_barrier()

        start = (4 * s - d0 + 1) * 64
        reads = [pltpu.async_copy(xs.at[pl.ds(start, PPW)], stx, sem),
                 pltpu.async_copy(ys.at[pl.ds(start, PPW)], sty, sem)]
        for cp in reads:
            cp.wait()

        def row_cand(k, _):
            sl = pl.ds(k * 16, 16)
            cxb[sl] = _clipx(stx[sl])
            cyb[sl] = _clipy(sty[sl] + d0f)
            return 0

        lax.fori_loop(0, NCH, row_cand, 0)
        eval_cand(init=False)

        def col_cand(k, _):
            sl = pl.ds(k * 16, 16)
            rb = (k >> 2) * 64
            kc = k & 3
            prev_sl = pl.ds(rb + ((kc - 1) & 3) * 16, 16)
            next_sl = pl.ds(rb + ((kc + 1) & 3) * 16, 16)
            d1p = (d1 > 0).astype(jnp.float32)
            lv = lax.iota(jnp.int32, 16)
            perm = (lv - d1) & 15

            dnums = lax.GatherDimensionNumbers(
                offset_dims=(), collapsed_slice_dims=(0,),
                start_index_map=(0,))

            def vperm(vec):
                return lax.gather(
                    vec, perm[:, None], dnums, (1,),
                    mode=lax.GatherScatterMode.PROMISE_IN_BOUNDS)

            def shifted(buf):
                rc = vperm(buf[sl])
                rp = vperm(buf[prev_sl])
                rn = vperm(buf[next_sl])
                right = jnp.where(lv == 0, rp, rc)
                left = jnp.where(lv == 15, rn, rc)
                return right * d1p + left * (1.0 - d1p)

            cxb[sl] = _clipx(shifted(xbuf) + d1f)
            cyb[sl] = _clipy(shifted(ybuf))
            return 0

        lax.fori_loop(0, NCH, col_cand, 0)
        eval_cand(init=False)

        @pl.when(r < 3)
        def _():
            def rs_cand(k, _):
                sl = pl.ds(k * 16, 16)
                cxb[sl] = _clipx(xbuf[sl] + nzv[2 * r, sl])
                cyb[sl] = _clipy(ybuf[sl] + nzv[2 * r + 1, sl])
                return 0

            lax.fori_loop(0, NCH, rs_cand, 0)
            eval_cand(init=False)

        return 0

    lax.fori_loop(0, 4, round_body, 0)

    outs = [pltpu.async_copy(xbuf, res_hbm.at[c, 0, pl.ds(base, PPW)], sem),
            pltpu.async_copy(ybuf, res_hbm.at[c, 1, pl.ds(base, PPW)], sem)]
    for cp in outs:
        cp.wait()


def _fusion_body(res_flat_hbm, mf_hbm, out_hbm,
                 rfx, rfy, mfx, mfy, ox, oy, wxb, wyb, idxb, taps, sem):
    c = lax.axis_index("c")
    s = lax.axis_index("s")
    wid = s * 2 + c
    base = wid * 128

    loads = [pltpu.async_copy(res_flat_hbm.at[pl.ds(base, 128)], rfx, sem),
             pltpu.async_copy(res_flat_hbm.at[pl.ds(N + base, 128)], rfy, sem),
             pltpu.async_copy(mf_hbm.at[0, pl.ds(base, 128)], mfx, sem),
             pltpu.async_copy(mf_hbm.at[1, pl.ds(base, 128)], mfy, sem)]
    for cp in loads:
        cp.wait()

    def build(k, _):
        sl = pl.ds(k * 16, 16)
        x0, y0, x1, y1, wx, wy = _bilinear_prep(rfx[sl], rfy[sl])
        wxb[sl] = wx
        wyb[sl] = wy
        colo = pl.ds(k * 16, 16)
        for tap, (yy, xx) in enumerate(((y0, x0), (y0, x1),
                                        (y1, x0), (y1, x1))):
            i = yy * W + xx
            idxb[tap, colo] = 2 * N + i
            idxb[4 + tap, colo] = 3 * N + i
        return 0

    lax.fori_loop(0, 8, build, 0)

    copies = [pltpu.async_copy(res_flat_hbm.at[idxb.at[r]], taps.at[r], sem)
              for r in range(8)]
    for cp in copies:
        cp.wait()

    def chunk(k, _):
        sl = pl.ds(k * 16, 16)
        fx = rfx[sl]
        fy = rfy[sl]
        wx = wxb[sl]
        wy = wyb[sl]
        cx = _bilinear_mix(taps[0, sl], taps[1, sl], taps[2, sl],
                           taps[3, sl], wx, wy)
        cy = _bilinear_mix(taps[4, sl], taps[5, sl], taps[6, sl],
                           taps[7, sl], wx, wy)
        diff = jnp.maximum(jnp.abs(fx - cx), jnp.abs(fy - cy))
        invalid = diff > EPS
        ox[sl] = jnp.where(invalid, mfx[sl], fx)
        oy[sl] = jnp.where(invalid, mfy[sl], fy)
        return 0

    lax.fori_loop(0, 8, chunk, 0)

    wr = [pltpu.async_copy(ox, out_hbm.at[0, pl.ds(base, 128)], sem),
          pltpu.async_copy(oy, out_hbm.at[1, pl.ds(base, 128)], sem)]
    for cp in wr:
        cp.wait()


_DEPAD_BLK = 256


def _depad_body(i_ref, o_ref):
    v = i_ref[...]
    top = v[:, 0:32, :]
    bot = v[:, 32:64, :]
    o = jnp.concatenate([top, bot], axis=-1)
    o_ref[...] = o.reshape(_DEPAD_BLK * 32, 128)


def _depad(corr_map):
    corr3 = corr_map.reshape(N, H, W)
    grid = N // _DEPAD_BLK
    out = pl.pallas_call(
        _depad_body,
        grid=(grid,),
        in_specs=[pl.BlockSpec((_DEPAD_BLK, H, W), lambda i: (i, 0, 0))],
        out_specs=pl.BlockSpec((_DEPAD_BLK * 32, 128), lambda i: (i, 0)),
        out_shape=jax.ShapeDtypeStruct((N * 32, 128), jnp.float32),
    )(corr3)
    return out.reshape(N * N)


def _make_noise():
    key = jax.random.key(42)
    planes = []
    for h in range(2):
        kh = jax.random.fold_in(key, h)
        for t in range(3):
            n = R * jax.random.normal(jax.random.fold_in(kh, t),
                                      (1, H, W, 2), dtype=jnp.float32)
            n2 = n.reshape(N, 2).T
            planes.append(n2)
    return jnp.concatenate(planes).reshape(2, 6, N)


_NOISE_CACHE = []
try:
    with jax.default_device(jax.local_devices(backend="cpu")[0]):
        _NOISE_CACHE.append(np.asarray(_make_noise()))
except Exception:
    pass


def _noise_const():
    if not _NOISE_CACHE:
        try:
            with jax.default_device(jax.local_devices(backend="cpu")[0]):
                _NOISE_CACHE.append(np.asarray(_make_noise()))
        except Exception:
            return _make_noise()
    return jnp.asarray(_NOISE_CACHE[0])


def kernel(matching_f, matching_b, corr_map):
    mesh = plsc.VectorSubcoreMesh(core_axis_name="c", subcore_axis_name="s")

    propagate = functools.partial(
        pl.kernel,
        mesh=mesh,
        out_type=jax.ShapeDtypeStruct((2, 2, N), jnp.float32),
        scratch_types=[
            pltpu.VMEM((PPW,), jnp.float32),
            pltpu.VMEM((PPW,), jnp.float32),
            pltpu.VMEM((PPW,), jnp.float32),
            pltpu.VMEM((PPW,), jnp.float32),
            pltpu.VMEM((PPW,), jnp.float32),
            pltpu.VMEM((PPW,), jnp.float32),
            pltpu.VMEM((PPW,), jnp.float32),
            pltpu.VMEM((PPW,), jnp.float32),
            pltpu.VMEM((PPW,), jnp.float32),
            pltpu.VMEM((6, PPW), jnp.float32),
            pltpu.VMEM((8, 128), jnp.int32),
            pltpu.VMEM((8, 128), jnp.float32),
            pltpu.VMEM_SHARED((66 * 64,), jnp.float32),
            pltpu.VMEM_SHARED((66 * 64,), jnp.float32),
            pltpu.SemaphoreType.DMA,
        ],
    )(_propagate_body)

    fusion = functools.partial(
        pl.kernel,
        mesh=mesh,
        out_type=jax.ShapeDtypeStruct((2, N), jnp.float32),
        scratch_types=[
            pltpu.VMEM((128,), jnp.float32),
            pltpu.VMEM((128,), jnp.float32),
            pltpu.VMEM((128,), jnp.float32),
            pltpu.VMEM((128,), jnp.float32),
            pltpu.VMEM((128,), jnp.float32),
            pltpu.VMEM((128,), jnp.float32),
            pltpu.VMEM((128,), jnp.float32),
            pltpu.VMEM((128,), jnp.float32),
            pltpu.VMEM((8, 128), jnp.int32),
            pltpu.VMEM((8, 128), jnp.float32),
            pltpu.SemaphoreType.DMA,
        ],
    )(_fusion_body)

    m_init = jnp.stack([matching_f.reshape(2, N), matching_b.reshape(2, N)])
    corr_flat = _depad(corr_map)
    noise = _noise_const()

    res = propagate(m_init, corr_flat, noise)
    out = fusion(res.reshape(4 * N), matching_f.reshape(2, N))
    return out.reshape(1, 2, H, W)
